# baseline (speedup 1.0000x reference)
import jax
import jax.numpy as jnp
from jax.experimental import pallas as pl
from jax.experimental.pallas import tpu as pltpu

_N, _D = 100000, 128
_BLOCK = 10000
_CHUNK = 1250


def _node_kernel(n0_ref, n1_ref, n2_ref, feat_ref, sums_ref):
    i = pl.program_id(0)
    p = pl.program_id(1)

    @pl.when(jnp.logical_and(i == 0, p == 0))
    def _():
        sums_ref[0] = 0.0
        sums_ref[1] = 0.0

    def body(x_ref, y_ref):
        acc = jnp.float32(0.0)
        for k in range(_BLOCK // _CHUNK):
            rows = pl.ds(k * _CHUNK, _CHUNK)
            x = x_ref[rows, :]
            y = y_ref[rows, :]
            prod = x * y
            feat_ref[0, rows, 0:_D] = x
            feat_ref[0, rows, _D:2 * _D] = y
            feat_ref[0, rows, 2 * _D:3 * _D] = prod
            acc += jnp.sum(prod)
        return acc

    @pl.when(p == 0)
    def _():
        sums_ref[0] += body(n0_ref, n1_ref)

    @pl.when(p == 1)
    def _():
        sums_ref[1] += body(n1_ref, n2_ref)


def kernel(node0, node1, node2):
    n = node0.shape[0]
    feats, sums = pl.pallas_call(
        _node_kernel,
        grid=(n // _BLOCK, 2),
        in_specs=[
            pl.BlockSpec((_BLOCK, _D), lambda i, p: (i, 0)),
            pl.BlockSpec((_BLOCK, _D), lambda i, p: (i, 0)),
            pl.BlockSpec((_BLOCK, _D), lambda i, p: (i, 0)),
        ],
        out_specs=[
            pl.BlockSpec((1, _BLOCK, 3 * _D), lambda i, p: (p, i, 0)),
            pl.BlockSpec(memory_space=pltpu.SMEM),
        ],
        out_shape=[
            jax.ShapeDtypeStruct((2, n, 3 * _D), jnp.float32),
            jax.ShapeDtypeStruct((2,), jnp.float32),
        ],
        compiler_params=pltpu.CompilerParams(vmem_limit_bytes=66060288),
    )(node0, node1, node2)
    return feats, sums


# speedup vs baseline: 1.0088x; 1.0088x over previous
import jax
import jax.numpy as jnp
from jax.experimental import pallas as pl
from jax.experimental.pallas import tpu as pltpu

_N, _D = 100000, 128
_BLOCK = 5000
_CHUNK = 1000


def _node_kernel(n0_ref, n1_ref, n2_ref, feat_ref, sums_ref):
    i = pl.program_id(0)

    @pl.when(i == 0)
    def _():
        sums_ref[0] = 0.0
        sums_ref[1] = 0.0

    acc01 = jnp.float32(0.0)
    acc12 = jnp.float32(0.0)
    for k in range(_BLOCK // _CHUNK):
        rows = pl.ds(k * _CHUNK, _CHUNK)
        a = n0_ref[rows, :]
        b = n1_ref[rows, :]
        c = n2_ref[rows, :]
        p01 = a * b
        p12 = b * c
        feat_ref[0, rows, 0:_D] = a
        feat_ref[0, rows, _D:2 * _D] = b
        feat_ref[0, rows, 2 * _D:3 * _D] = p01
        feat_ref[1, rows, 0:_D] = b
        feat_ref[1, rows, _D:2 * _D] = c
        feat_ref[1, rows, 2 * _D:3 * _D] = p12
        acc01 += jnp.sum(p01)
        acc12 += jnp.sum(p12)
    sums_ref[0] += acc01
    sums_ref[1] += acc12


def kernel(node0, node1, node2):
    n = node0.shape[0]
    feats, sums = pl.pallas_call(
        _node_kernel,
        grid=(n // _BLOCK,),
        in_specs=[
            pl.BlockSpec((_BLOCK, _D), lambda i: (i, 0)),
            pl.BlockSpec((_BLOCK, _D), lambda i: (i, 0)),
            pl.BlockSpec((_BLOCK, _D), lambda i: (i, 0)),
        ],
        out_specs=[
            pl.BlockSpec((2, _BLOCK, 3 * _D), lambda i: (0, i, 0)),
            pl.BlockSpec(memory_space=pltpu.SMEM),
        ],
        out_shape=[
            jax.ShapeDtypeStruct((2, n, 3 * _D), jnp.float32),
            jax.ShapeDtypeStruct((2,), jnp.float32),
        ],
    )(node0, node1, node2)
    return feats, sums


# speedup vs baseline: 1.0108x; 1.0020x over previous
import jax
import jax.numpy as jnp
from jax.experimental import pallas as pl
from jax.experimental.pallas import tpu as pltpu

_D = 128
_BLOCK = 5000


def _node_kernel(n0_ref, n1_ref, n2_ref, feat_ref, sums_ref):
    i = pl.program_id(0)
    a = n0_ref[...]
    b = n1_ref[...]
    c = n2_ref[...]
    p01 = a * b
    p12 = b * c
    feat_ref[0, :, 0:_D] = a
    feat_ref[0, :, _D:2 * _D] = b
    feat_ref[0, :, 2 * _D:3 * _D] = p01
    feat_ref[1, :, 0:_D] = b
    feat_ref[1, :, _D:2 * _D] = c
    feat_ref[1, :, 2 * _D:3 * _D] = p12

    @pl.when(i == 0)
    def _():
        sums_ref[0] = 0.0
        sums_ref[1] = 0.0

    sums_ref[0] += jnp.sum(p01)
    sums_ref[1] += jnp.sum(p12)


def kernel(node0, node1, node2):
    n = node0.shape[0]
    feats, sums = pl.pallas_call(
        _node_kernel,
        grid=(n // _BLOCK,),
        in_specs=[
            pl.BlockSpec((_BLOCK, _D), lambda i: (i, 0)),
            pl.BlockSpec((_BLOCK, _D), lambda i: (i, 0)),
            pl.BlockSpec((_BLOCK, _D), lambda i: (i, 0)),
        ],
        out_specs=[
            pl.BlockSpec((2, _BLOCK, 3 * _D), lambda i: (0, i, 0)),
            pl.BlockSpec(memory_space=pltpu.SMEM),
        ],
        out_shape=[
            jax.ShapeDtypeStruct((2, n, 3 * _D), jnp.float32),
            jax.ShapeDtypeStruct((2,), jnp.float32),
        ],
    )(node0, node1, node2)
    return feats, sums
